# baseline (speedup 1.0000x reference)
import functools

import jax
import jax.numpy as jnp
from jax.experimental import pallas as pl

N = 10000
E = 160000
D_IN = 256
HID = 512
HEADS = 2

_BM = 2000


def _mm_att_body(x_ref, w_ref, a_ref, h_ref, att_ref):
    h = jnp.dot(x_ref[...], w_ref[...], preferred_element_type=jnp.float32)
    h_ref[...] = h
    att_ref[...] = jnp.dot(h, a_ref[...], preferred_element_type=jnp.float32)


def _mm_att(x, w, a):
    n, k = x.shape
    f = w.shape[1]
    p = a.shape[1]
    grid = n // _BM
    return pl.pallas_call(
        _mm_att_body,
        grid=(grid,),
        in_specs=[
            pl.BlockSpec((_BM, k), lambda i: (i, 0)),
            pl.BlockSpec((k, f), lambda i: (0, 0)),
            pl.BlockSpec((f, p), lambda i: (0, 0)),
        ],
        out_specs=[
            pl.BlockSpec((_BM, f), lambda i: (i, 0)),
            pl.BlockSpec((_BM, p), lambda i: (i, 0)),
        ],
        out_shape=[
            jax.ShapeDtypeStruct((n, f), jnp.float32),
            jax.ShapeDtypeStruct((n, p), jnp.float32),
        ],
    )(x, w, a)


def _elu_mm_att_body(g_ref, b_ref, w_ref, a_ref, h_ref, att_ref):
    x = g_ref[...] + b_ref[...]
    x = jnp.where(x > 0, x, jnp.exp(jnp.minimum(x, 0.0)) - 1.0)
    h = jnp.dot(x, w_ref[...], preferred_element_type=jnp.float32)
    h_ref[...] = h
    att_ref[...] = jnp.dot(h, a_ref[...], preferred_element_type=jnp.float32)


def _elu_mm_att(g, b, w, a):
    n, k = g.shape
    f = w.shape[1]
    p = a.shape[1]
    grid = n // _BM
    return pl.pallas_call(
        _elu_mm_att_body,
        grid=(grid,),
        in_specs=[
            pl.BlockSpec((_BM, k), lambda i: (i, 0)),
            pl.BlockSpec((1, k), lambda i: (0, 0)),
            pl.BlockSpec((k, f), lambda i: (0, 0)),
            pl.BlockSpec((f, p), lambda i: (0, 0)),
        ],
        out_specs=[
            pl.BlockSpec((_BM, f), lambda i: (i, 0)),
            pl.BlockSpec((_BM, p), lambda i: (i, 0)),
        ],
        out_shape=[
            jax.ShapeDtypeStruct((n, f), jnp.float32),
            jax.ShapeDtypeStruct((n, p), jnp.float32),
        ],
    )(g, b[None, :], w, a)


def _edge_softmax_agg(h, a_src, a_dst, src, dst, heads, ch):
    n = h.shape[0]
    e = a_src[src] + a_dst[dst]
    e = jax.nn.leaky_relu(e, negative_slope=0.2)
    e_max = jax.ops.segment_max(e, dst, num_segments=n)
    e_max = jnp.where(jnp.isfinite(e_max), e_max, 0.0)
    p = jnp.exp(e - e_max[dst])
    denom = jax.ops.segment_sum(p, dst, num_segments=n)
    alpha = p / (denom[dst] + 1e-16)
    hh = h.reshape(n, heads, ch)
    msg = hh[src] * alpha[:, :, None]
    out = jax.ops.segment_sum(msg, dst, num_segments=n)
    return out.reshape(n, heads * ch)


def kernel(x, edge_index, W1, att_src1, att_dst1, b1, W2, att_src2, att_dst2, b2, W_out, b_out):
    src = edge_index[0]
    dst = edge_index[1]

    z = jnp.zeros((HID,), jnp.float32)
    asrc1 = att_src1.reshape(HEADS, HID)
    adst1 = att_dst1.reshape(HEADS, HID)
    A1 = jnp.stack(
        [
            jnp.concatenate([asrc1[0], z]),
            jnp.concatenate([z, asrc1[1]]),
            jnp.concatenate([adst1[0], z]),
            jnp.concatenate([z, adst1[1]]),
        ],
        axis=1,
    )
    A2 = jnp.stack(
        [att_src2.reshape(HID), att_dst2.reshape(HID)], axis=1
    )

    h1, att1 = _mm_att(x, W1, A1)
    a_src_1 = att1[:, :HEADS]
    a_dst_1 = att1[:, HEADS:]
    out1 = _edge_softmax_agg(h1, a_src_1, a_dst_1, src, dst, HEADS, HID)

    h2, att2 = _elu_mm_att(out1, b1, W2, A2)
    a_src_2 = att2[:, :1]
    a_dst_2 = att2[:, 1:]
    out2 = _edge_softmax_agg(h2, a_src_2, a_dst_2, src, dst, 1, HID)

    Wb = jnp.concatenate([W_out, jnp.zeros((HID, 1), jnp.float32)], axis=1)
    y, _ = _elu_mm_att(out2, b2, Wb, jnp.zeros((2, 1), jnp.float32))
    return y[:, :1] + b_out


# speedup vs baseline: 1.1777x; 1.1777x over previous
import functools

import jax
import jax.numpy as jnp
from jax import lax
from jax.experimental import pallas as pl
from jax.experimental.pallas import tpu as pltpu
from jax.experimental.pallas import tpu_sc as plsc

N = 10000
E = 160000
D_IN = 256
HID = 512
HEADS = 2

S = 10240
E_PAD = 160256
EPT = E_PAD // 32
VPT = EPT // 16
NEG = -3.0e38

_BM = 2000
_BM2 = 1280

_mesh = lambda: plsc.VectorSubcoreMesh(
    core_axis_name="c", subcore_axis_name="s", num_cores=2, num_subcores=16)
_SC_PARAMS = pltpu.CompilerParams(needs_layout_passes=False)


def _wid():
    return lax.axis_index("s") * 2 + lax.axis_index("c")


def _mm_att_body(x_ref, w_ref, a_ref, h_ref, att_ref):
    h = jnp.dot(x_ref[...], w_ref[...], preferred_element_type=jnp.float32)
    h_ref[...] = h
    att_ref[...] = jnp.dot(h, a_ref[...], preferred_element_type=jnp.float32)


def _mm_att(x, w, a):
    n, k = x.shape
    f = w.shape[1]
    p = a.shape[1]
    return pl.pallas_call(
        _mm_att_body,
        grid=(n // _BM,),
        in_specs=[
            pl.BlockSpec((_BM, k), lambda i: (i, 0)),
            pl.BlockSpec((k, f), lambda i: (0, 0)),
            pl.BlockSpec((f, p), lambda i: (0, 0)),
        ],
        out_specs=[
            pl.BlockSpec((_BM, f), lambda i: (i, 0)),
            pl.BlockSpec((_BM, p), lambda i: (i, 0)),
        ],
        out_shape=[
            jax.ShapeDtypeStruct((n, f), jnp.float32),
            jax.ShapeDtypeStruct((n, p), jnp.float32),
        ],
    )(x, w, a)


def _elu_mm_att_body(g_ref, b_ref, w_ref, a_ref, h_ref, att_ref):
    x = g_ref[...] + b_ref[...]
    x = jnp.where(x > 0, x, jnp.exp(jnp.minimum(x, 0.0)) - 1.0)
    h = jnp.dot(x, w_ref[...], preferred_element_type=jnp.float32)
    h_ref[...] = h
    att_ref[...] = jnp.dot(h, a_ref[...], preferred_element_type=jnp.float32)


def _elu_mm_att(g, b, w, a):
    n, k = g.shape
    f = w.shape[1]
    p = a.shape[1]
    return pl.pallas_call(
        _elu_mm_att_body,
        grid=(n // _BM2,),
        in_specs=[
            pl.BlockSpec((_BM2, k), lambda i: (i, 0)),
            pl.BlockSpec((1, k), lambda i: (0, 0)),
            pl.BlockSpec((k, f), lambda i: (0, 0)),
            pl.BlockSpec((f, p), lambda i: (0, 0)),
        ],
        out_specs=[
            pl.BlockSpec((_BM2, f), lambda i: (i, 0)),
            pl.BlockSpec((_BM2, p), lambda i: (i, 0)),
        ],
        out_shape=[
            jax.ShapeDtypeStruct((n, f), jnp.float32),
            jax.ShapeDtypeStruct((n, p), jnp.float32),
        ],
    )(g, b[None, :], w, a)


def _monotone_u32(e):
    bits = plsc.bitcast(e, jnp.uint32)
    mneg = jnp.uint32(0) - (bits >> 31)
    return bits ^ (mneg | jnp.uint32(0x80000000))


def _make_phase_a(K):
    KN = K * S

    @functools.partial(
        pl.kernel,
        out_type=jax.ShapeDtypeStruct((32 * KN,), jnp.float32),
        mesh=_mesh(),
        compiler_params=_SC_PARAMS,
        scratch_types=[
            pltpu.VMEM((EPT,), jnp.int32),
            pltpu.VMEM((EPT,), jnp.int32),
            pltpu.VMEM((KN,), jnp.float32),
            pltpu.VMEM((KN,), jnp.float32),
            pltpu.VMEM((KN,), jnp.float32),
        ],
    )
    def phase_a(src_hbm, dst_hbm, asrc_hbm, adst_hbm, mpart_hbm,
                src_v, dst_v, asrc_v, adst_v, m_v):
        wid = _wid()
        base = wid * EPT
        pltpu.sync_copy(src_hbm.at[pl.ds(base, EPT)], src_v)
        pltpu.sync_copy(dst_hbm.at[pl.ds(base, EPT)], dst_v)
        pltpu.sync_copy(asrc_hbm, asrc_v)
        pltpu.sync_copy(adst_hbm, adst_v)

        def init(i, _):
            m_v[pl.ds(i * 16, 16)] = jnp.full((16,), NEG, jnp.float32)
            return 0

        lax.fori_loop(0, KN // 16, init, 0)

        def body(v, _):
            sl = pl.ds(v * 16, 16)
            sv = src_v[sl]
            dv = dst_v[sl]
            for k in range(K):
                off = k * S
                a_s = plsc.load_gather(asrc_v, [sv + off])
                a_d = plsc.load_gather(adst_v, [dv + off])
                x = a_s + a_d
                e = jnp.maximum(x, 0.2 * x)
                key = (dv.astype(jnp.uint32) << 18) | (_monotone_u32(e) >> 14)
                key_s, e_s = plsc.sort_key_val(key, e)
                d_s = (key_s >> 18).astype(jnp.int32)
                _, last = plsc.scan_count(d_s)
                idx = d_s + off
                cur = plsc.load_gather(m_v, [idx])
                plsc.store_scatter(m_v, [idx], jnp.maximum(cur, e_s), mask=last)
            return 0

        lax.fori_loop(0, VPT, body, 0)
        pltpu.sync_copy(m_v, mpart_hbm.at[pl.ds(wid * KN, KN)])

    return phase_a


def _make_merge(K, add):
    KN = K * S
    CH = KN // 32

    @functools.partial(
        pl.kernel,
        out_type=jax.ShapeDtypeStruct((KN,), jnp.float32),
        mesh=_mesh(),
        compiler_params=_SC_PARAMS,
        scratch_types=[
            pltpu.VMEM((32 * CH,), jnp.float32),
            pltpu.VMEM((CH,), jnp.float32),
        ],
    )
    def merge(part_hbm, glob_hbm, buf, outv):
        wid = _wid()
        for t in range(32):
            pltpu.sync_copy(part_hbm.at[pl.ds(t * KN + wid * CH, CH)],
                            buf.at[pl.ds(t * CH, CH)])

        def body(i, _):
            acc = buf[pl.ds(i * 16, 16)]
            for t in range(1, 32):
                x = buf[pl.ds(t * CH + i * 16, 16)]
                acc = acc + x if add else jnp.maximum(acc, x)
            outv[pl.ds(i * 16, 16)] = acc
            return 0

        lax.fori_loop(0, CH // 16, body, 0)
        pltpu.sync_copy(outv, glob_hbm.at[pl.ds(wid * CH, CH)])

    return merge


def _make_phase_c(K):
    KN = K * S

    @functools.partial(
        pl.kernel,
        out_type=[
            jax.ShapeDtypeStruct((K * E_PAD,), jnp.float32),
            jax.ShapeDtypeStruct((32 * KN,), jnp.float32),
        ],
        mesh=_mesh(),
        compiler_params=_SC_PARAMS,
        scratch_types=[
            pltpu.VMEM((EPT,), jnp.int32),
            pltpu.VMEM((EPT,), jnp.int32),
            pltpu.VMEM((KN,), jnp.float32),
            pltpu.VMEM((KN,), jnp.float32),
            pltpu.VMEM((KN,), jnp.float32),
            pltpu.VMEM((KN,), jnp.float32),
            pltpu.VMEM((K * EPT,), jnp.float32),
        ],
    )
    def phase_c(src_hbm, dst_hbm, asrc_hbm, adst_hbm, mglob_hbm,
                p_hbm, denpart_hbm,
                src_v, dst_v, asrc_v, adst_v, mg_v, den_v, p_v):
        wid = _wid()
        base = wid * EPT
        pltpu.sync_copy(src_hbm.at[pl.ds(base, EPT)], src_v)
        pltpu.sync_copy(dst_hbm.at[pl.ds(base, EPT)], dst_v)
        pltpu.sync_copy(asrc_hbm, asrc_v)
        pltpu.sync_copy(adst_hbm, adst_v)
        pltpu.sync_copy(mglob_hbm, mg_v)

        def init(i, _):
            den_v[pl.ds(i * 16, 16)] = jnp.zeros((16,), jnp.float32)
            return 0

        lax.fori_loop(0, KN // 16, init, 0)

        def body(v, _):
            sl = pl.ds(v * 16, 16)
            sv = src_v[sl]
            dv = dst_v[sl]
            for k in range(K):
                off = k * S
                a_s = plsc.load_gather(asrc_v, [sv + off])
                a_d = plsc.load_gather(adst_v, [dv + off])
                x = a_s + a_d
                e = jnp.maximum(x, 0.2 * x)
                m_d = plsc.load_gather(mg_v, [dv + off])
                p = jnp.exp(e - m_d)
                p_v[pl.ds(k * EPT + v * 16, 16)] = p
                plsc.addupdate_scatter(den_v, [dv + off], p)
            return 0

        lax.fori_loop(0, VPT, body, 0)
        for k in range(K):
            pltpu.sync_copy(p_v.at[pl.ds(k * EPT, EPT)],
                            p_hbm.at[pl.ds(k * E_PAD + base, EPT)])
        pltpu.sync_copy(den_v, denpart_hbm.at[pl.ds(wid * KN, KN)])

    return phase_c


def _make_phase_e(K, F, P):
    C = F // K
    CCH = C // 16
    RPT = S // 32
    RPP = RPT // P
    ACC = (RPP + 1) * F
    CHK = EPT
    NCHK = E_PAD // CHK

    @functools.partial(
        pl.kernel,
        out_type=jax.ShapeDtypeStruct((S * F,), jnp.float32),
        mesh=_mesh(),
        compiler_params=_SC_PARAMS,
        scratch_types=[
            pltpu.VMEM((CHK,), jnp.int32),
            pltpu.VMEM((CHK,), jnp.int32),
            pltpu.VMEM((K * CHK,), jnp.float32),
            pltpu.VMEM((CHK + 16,), jnp.int32),
            pltpu.VMEM((16, F), jnp.float32),
            pltpu.VMEM((K * RPP,), jnp.float32),
            pltpu.VMEM((ACC,), jnp.float32),
            pltpu.SemaphoreType.DMA,
        ],
    )
    def phase_e(dst_hbm, src_hbm, p_hbm, h_hbm, den_hbm, outf_hbm,
                dch, sch, pch, posb, rowb, denb, acc, sem3):
        wid = _wid()
        ji = lax.iota(jnp.int32, 16)

        for pi in range(P):
            row_lo = wid * RPT + pi * RPP

            def zero(i, _):
                acc[pl.ds(i * 16, 16)] = jnp.zeros((16,), jnp.float32)
                return 0

            lax.fori_loop(0, ACC // 16, zero, 0)

            def chunk(cc32, _):
                pltpu.sync_copy(dst_hbm.at[pl.ds(cc32 * CHK, CHK)], dch)
                pltpu.sync_copy(src_hbm.at[pl.ds(cc32 * CHK, CHK)], sch)
                for k in range(K):
                    pltpu.sync_copy(
                        p_hbm.at[pl.ds(k * E_PAD + cc32 * CHK, CHK)],
                        pch.at[pl.ds(k * CHK, CHK)])

                def sel(v, cnt):
                    dv = dch[pl.ds(v * 16, 16)]
                    msk = (dv >= row_lo) & (dv < row_lo + RPP)
                    cs = plsc.cumsum(msk.astype(jnp.int32))
                    plsc.store_scatter(posb, [cnt + cs - 1],
                                       ji + v * 16, mask=msk)
                    return cnt + jnp.max(cs)

                cnt = lax.fori_loop(0, VPT, sel, 0)
                ng = (cnt + 15) // 16

                def grp(g, _):
                    pos = posb[pl.ds(g * 16, 16)]
                    valid = (g * 16 + ji) < cnt
                    pos = jnp.where(valid, pos, 0)
                    dv = plsc.load_gather(dch, [pos])
                    d_loc = jnp.where(valid, dv - row_lo, RPP)
                    dbase = d_loc * F
                    sv = plsc.load_gather(sch, [pos])
                    sv = jnp.where(valid, sv, wid)
                    cp3 = pltpu.async_copy(h_hbm.at[sv], rowb, sem3)
                    alphas = [jnp.where(valid,
                                        plsc.load_gather(pch, [pos + k * CHK]),
                                        0.0)
                              for k in range(K)]
                    cp3.wait()
                    for j in range(16):
                        dj = jnp.max(jnp.where(ji == j, dbase, 0))
                        for k in range(K):
                            spl = jnp.max(jnp.where(ji == j, alphas[k], 0.0))

                            def accum(cc, _, dj=dj, k=k, j=j, spl=spl):
                                off = dj + k * C + cc * 16
                                acc[pl.ds(off, 16)] = (
                                    acc[pl.ds(off, 16)]
                                    + spl * rowb[j, pl.ds(k * C + cc * 16, 16)])
                                return 0

                            lax.fori_loop(0, CCH, accum, 0)
                    return 0

                lax.fori_loop(0, ng, grp, 0)
                return 0

            lax.fori_loop(0, NCHK, chunk, 0)

            for k in range(K):
                pltpu.sync_copy(den_hbm.at[pl.ds(k * S + row_lo, RPP)],
                                denb.at[pl.ds(k * RPP, RPP)])

            def drain(q, _):
                for k in range(K):
                    den16 = denb[pl.ds(k * RPP + q * 16, 16)]
                    rec = 1.0 / (den16 + 1e-16)
                    for j in range(16):
                        rs = jnp.max(jnp.where(ji == j, rec, 0.0))
                        rbase = (q * 16 + j) * F + k * C

                        def dsc(cc, _, rbase=rbase, rs=rs):
                            off = rbase + cc * 16
                            acc[pl.ds(off, 16)] = acc[pl.ds(off, 16)] * rs
                            return 0

                        lax.fori_loop(0, CCH, dsc, 0)
                return 0

            lax.fori_loop(0, RPP // 16, drain, 0)
            pltpu.sync_copy(acc.at[pl.ds(0, RPP * F)],
                            outf_hbm.at[pl.ds(row_lo * F, RPP * F)])

    return phase_e


def _node_table(cols):
    k = cols.shape[1]
    tab = jnp.zeros((k, S), jnp.float32)
    tab = tab.at[:, :N].set(cols.T)
    return tab.reshape(-1)


def _gat_edge_phase(src_p, dst_p, a_src, a_dst, h, K, F, P):
    asrc_f = _node_table(a_src)
    adst_f = _node_table(a_dst)
    m_part = _make_phase_a(K)(src_p, dst_p, asrc_f, adst_f)
    m_glob = _make_merge(K, add=False)(m_part)
    p_all, den_part = _make_phase_c(K)(src_p, dst_p, asrc_f, adst_f, m_glob)
    den_glob = _make_merge(K, add=True)(den_part)
    if True:
        pmat = p_all.reshape(K, E_PAD)[:, :E].T
        den = den_glob.reshape(K, S)
        alpha = pmat / (den[:, dst_p[:E]].T + 1e-16)
        hh = h[:N].reshape(N, K, F // K)
        msg = hh[src_p[:E]] * alpha[:, :, None]
        out = jax.ops.segment_sum(msg, dst_p[:E], num_segments=N)
        out = out.reshape(N, F)
        return jnp.concatenate([out, jnp.zeros((S - N, F))], axis=0)
    outf = _make_phase_e(K, F, P)(dst_p, src_p, p_all, h, den_glob)
    return outf.reshape(S, F)


def kernel(x, edge_index, W1, att_src1, att_dst1, b1, W2, att_src2, att_dst2, b2, W_out, b_out):
    src = edge_index[0]
    dst = edge_index[1]
    pad = E_PAD - E
    src_p = jnp.concatenate([src, jnp.zeros((pad,), jnp.int32)])
    dst_p = jnp.concatenate([dst, jnp.full((pad,), N, jnp.int32)])

    z = jnp.zeros((HID,), jnp.float32)
    asrc1 = att_src1.reshape(HEADS, HID)
    adst1 = att_dst1.reshape(HEADS, HID)
    A1 = jnp.stack(
        [
            jnp.concatenate([asrc1[0], z]),
            jnp.concatenate([z, asrc1[1]]),
            jnp.concatenate([adst1[0], z]),
            jnp.concatenate([z, adst1[1]]),
        ],
        axis=1,
    )
    A2 = jnp.stack([att_src2.reshape(HID), att_dst2.reshape(HID)], axis=1)

    h1, att1 = _mm_att(x, W1, A1)
    out1 = _gat_edge_phase(src_p, dst_p, att1[:, :HEADS], att1[:, HEADS:],
                           h1, HEADS, HEADS * HID, 5)

    h2, att2 = _elu_mm_att(out1, b1, W2, A2)
    out2 = _gat_edge_phase(src_p, dst_p, att2[:N, :1], att2[:N, 1:],
                           h2, 1, HID, 2)

    Wb = jnp.concatenate([W_out, jnp.zeros((HID, 1), jnp.float32)], axis=1)
    y, _ = _elu_mm_att(out2, b2, Wb, jnp.zeros((2, 1), jnp.float32))
    return y[:N, :1] + b_out


# speedup vs baseline: 3.8718x; 3.2876x over previous
import functools

import jax
import jax.numpy as jnp
from jax import lax
from jax.experimental import pallas as pl
from jax.experimental.pallas import tpu as pltpu
from jax.experimental.pallas import tpu_sc as plsc

N = 10000
E = 160000
D_IN = 256
HID = 512
HEADS = 2

S = 10240
E_PAD = 160256
EPT = E_PAD // 32
VPT = EPT // 16
NEG = -3.0e38

_BM = 2000
_BM2 = 1280

_mesh = lambda: plsc.VectorSubcoreMesh(
    core_axis_name="c", subcore_axis_name="s", num_cores=2, num_subcores=16)
_SC_PARAMS = pltpu.CompilerParams(needs_layout_passes=False)


def _wid():
    return lax.axis_index("s") * 2 + lax.axis_index("c")


def _mm_att_body(x_ref, w_ref, a_ref, h_ref, att_ref):
    h = jnp.dot(x_ref[...], w_ref[...], preferred_element_type=jnp.float32)
    h_ref[...] = h
    att_ref[...] = jnp.dot(h, a_ref[...], preferred_element_type=jnp.float32)


def _mm_att(x, w, a):
    n, k = x.shape
    f = w.shape[1]
    p = a.shape[1]
    return pl.pallas_call(
        _mm_att_body,
        grid=(n // _BM,),
        in_specs=[
            pl.BlockSpec((_BM, k), lambda i: (i, 0)),
            pl.BlockSpec((k, f), lambda i: (0, 0)),
            pl.BlockSpec((f, p), lambda i: (0, 0)),
        ],
        out_specs=[
            pl.BlockSpec((_BM, f), lambda i: (i, 0)),
            pl.BlockSpec((_BM, p), lambda i: (i, 0)),
        ],
        out_shape=[
            jax.ShapeDtypeStruct((n, f), jnp.float32),
            jax.ShapeDtypeStruct((n, p), jnp.float32),
        ],
    )(x, w, a)


def _elu_mm_att_body(g_ref, b_ref, w_ref, a_ref, h_ref, att_ref):
    x = g_ref[...] + b_ref[...]
    x = jnp.where(x > 0, x, jnp.exp(jnp.minimum(x, 0.0)) - 1.0)
    h = jnp.dot(x, w_ref[...], preferred_element_type=jnp.float32)
    h_ref[...] = h
    att_ref[...] = jnp.dot(h, a_ref[...], preferred_element_type=jnp.float32)


def _elu_mm_att(g, b, w, a):
    n, k = g.shape
    f = w.shape[1]
    p = a.shape[1]
    return pl.pallas_call(
        _elu_mm_att_body,
        grid=(n // _BM2,),
        in_specs=[
            pl.BlockSpec((_BM2, k), lambda i: (i, 0)),
            pl.BlockSpec((1, k), lambda i: (0, 0)),
            pl.BlockSpec((k, f), lambda i: (0, 0)),
            pl.BlockSpec((f, p), lambda i: (0, 0)),
        ],
        out_specs=[
            pl.BlockSpec((_BM2, f), lambda i: (i, 0)),
            pl.BlockSpec((_BM2, p), lambda i: (i, 0)),
        ],
        out_shape=[
            jax.ShapeDtypeStruct((n, f), jnp.float32),
            jax.ShapeDtypeStruct((n, p), jnp.float32),
        ],
    )(g, b[None, :], w, a)


def _monotone_u32(e):
    bits = plsc.bitcast(e, jnp.uint32)
    mneg = jnp.uint32(0) - (bits >> 31)
    return bits ^ (mneg | jnp.uint32(0x80000000))


def _make_phase_a(K):
    KN = K * S

    @functools.partial(
        pl.kernel,
        out_type=jax.ShapeDtypeStruct((32 * KN,), jnp.float32),
        mesh=_mesh(),
        compiler_params=_SC_PARAMS,
        scratch_types=[
            pltpu.VMEM((EPT,), jnp.int32),
            pltpu.VMEM((EPT,), jnp.int32),
            pltpu.VMEM((KN,), jnp.float32),
            pltpu.VMEM((KN,), jnp.float32),
            pltpu.VMEM((KN,), jnp.float32),
        ],
    )
    def phase_a(src_hbm, dst_hbm, asrc_hbm, adst_hbm, mpart_hbm,
                src_v, dst_v, asrc_v, adst_v, m_v):
        wid = _wid()
        base = wid * EPT
        pltpu.sync_copy(src_hbm.at[pl.ds(base, EPT)], src_v)
        pltpu.sync_copy(dst_hbm.at[pl.ds(base, EPT)], dst_v)
        pltpu.sync_copy(asrc_hbm, asrc_v)
        pltpu.sync_copy(adst_hbm, adst_v)

        def init(i, _):
            m_v[pl.ds(i * 16, 16)] = jnp.full((16,), NEG, jnp.float32)
            return 0

        lax.fori_loop(0, KN // 16, init, 0)

        def body(v, _):
            sl = pl.ds(v * 16, 16)
            sv = src_v[sl]
            dv = dst_v[sl]
            for k in range(K):
                off = k * S
                a_s = plsc.load_gather(asrc_v, [sv + off])
                a_d = plsc.load_gather(adst_v, [dv + off])
                x = a_s + a_d
                e = jnp.maximum(x, 0.2 * x)
                key = (dv.astype(jnp.uint32) << 18) | (_monotone_u32(e) >> 14)
                key_s, e_s = plsc.sort_key_val(key, e)
                d_s = (key_s >> 18).astype(jnp.int32)
                _, last = plsc.scan_count(d_s)
                idx = d_s + off
                cur = plsc.load_gather(m_v, [idx])
                plsc.store_scatter(m_v, [idx], jnp.maximum(cur, e_s), mask=last)
            return 0

        lax.fori_loop(0, VPT, body, 0)
        pltpu.sync_copy(m_v, mpart_hbm.at[pl.ds(wid * KN, KN)])

    return phase_a


def _make_merge(K, add):
    KN = K * S
    CH = KN // 32

    @functools.partial(
        pl.kernel,
        out_type=jax.ShapeDtypeStruct((KN,), jnp.float32),
        mesh=_mesh(),
        compiler_params=_SC_PARAMS,
        scratch_types=[
            pltpu.VMEM((32 * CH,), jnp.float32),
            pltpu.VMEM((CH,), jnp.float32),
        ],
    )
    def merge(part_hbm, glob_hbm, buf, outv):
        wid = _wid()
        for t in range(32):
            pltpu.sync_copy(part_hbm.at[pl.ds(t * KN + wid * CH, CH)],
                            buf.at[pl.ds(t * CH, CH)])

        def body(i, _):
            acc = buf[pl.ds(i * 16, 16)]
            for t in range(1, 32):
                x = buf[pl.ds(t * CH + i * 16, 16)]
                acc = acc + x if add else jnp.maximum(acc, x)
            outv[pl.ds(i * 16, 16)] = acc
            return 0

        lax.fori_loop(0, CH // 16, body, 0)
        pltpu.sync_copy(outv, glob_hbm.at[pl.ds(wid * CH, CH)])

    return merge


def _make_phase_c(K):
    KN = K * S

    @functools.partial(
        pl.kernel,
        out_type=[
            jax.ShapeDtypeStruct((K * E_PAD,), jnp.float32),
            jax.ShapeDtypeStruct((32 * KN,), jnp.float32),
        ],
        mesh=_mesh(),
        compiler_params=_SC_PARAMS,
        scratch_types=[
            pltpu.VMEM((EPT,), jnp.int32),
            pltpu.VMEM((EPT,), jnp.int32),
            pltpu.VMEM((KN,), jnp.float32),
            pltpu.VMEM((KN,), jnp.float32),
            pltpu.VMEM((KN,), jnp.float32),
            pltpu.VMEM((KN,), jnp.float32),
            pltpu.VMEM((K * EPT,), jnp.float32),
        ],
    )
    def phase_c(src_hbm, dst_hbm, asrc_hbm, adst_hbm, mglob_hbm,
                p_hbm, denpart_hbm,
                src_v, dst_v, asrc_v, adst_v, mg_v, den_v, p_v):
        wid = _wid()
        base = wid * EPT
        pltpu.sync_copy(src_hbm.at[pl.ds(base, EPT)], src_v)
        pltpu.sync_copy(dst_hbm.at[pl.ds(base, EPT)], dst_v)
        pltpu.sync_copy(asrc_hbm, asrc_v)
        pltpu.sync_copy(adst_hbm, adst_v)
        pltpu.sync_copy(mglob_hbm, mg_v)

        def init(i, _):
            den_v[pl.ds(i * 16, 16)] = jnp.zeros((16,), jnp.float32)
            return 0

        lax.fori_loop(0, KN // 16, init, 0)

        def body(v, _):
            sl = pl.ds(v * 16, 16)
            sv = src_v[sl]
            dv = dst_v[sl]
            for k in range(K):
                off = k * S
                a_s = plsc.load_gather(asrc_v, [sv + off])
                a_d = plsc.load_gather(adst_v, [dv + off])
                x = a_s + a_d
                e = jnp.maximum(x, 0.2 * x)
                m_d = plsc.load_gather(mg_v, [dv + off])
                p = jnp.exp(e - m_d)
                p_v[pl.ds(k * EPT + v * 16, 16)] = p
                plsc.addupdate_scatter(den_v, [dv + off], p)
            return 0

        lax.fori_loop(0, VPT, body, 0)
        for k in range(K):
            pltpu.sync_copy(p_v.at[pl.ds(k * EPT, EPT)],
                            p_hbm.at[pl.ds(k * E_PAD + base, EPT)])
        pltpu.sync_copy(den_v, denpart_hbm.at[pl.ds(wid * KN, KN)])

    return phase_c


def _make_phase_e(K, F, P):
    C = F // K
    CCH = C // 16
    RPT = S // 32
    RPP = RPT // P
    ACC = (RPP + 1) * F
    CHK = EPT
    NCHK = E_PAD // CHK

    @functools.partial(
        pl.kernel,
        out_type=jax.ShapeDtypeStruct((S * F,), jnp.float32),
        mesh=_mesh(),
        compiler_params=_SC_PARAMS,
        scratch_types=(
            [pltpu.VMEM((CHK,), jnp.int32)] * 2
            + [pltpu.VMEM((CHK,), jnp.float32)] * K
            + [
                pltpu.VMEM((CHK + 16,), jnp.int32),
                pltpu.VMEM((16 * F,), jnp.float32),
                pltpu.VMEM((K * RPP,), jnp.float32),
                pltpu.VMEM((ACC,), jnp.float32),
                pltpu.SemaphoreType.DMA,
            ]
        ),
    )
    def phase_e(dst_hbm, src_hbm, p_hbm, h_hbm, den_hbm, outf_hbm,
                dch, sch, *rest):
        pchs = rest[:K]
        posb, rowb, denb, acc, sem3 = rest[K:]
        wid = _wid()
        ji = lax.iota(jnp.int32, 16)

        for pi in range(P):
            row_lo = wid * RPT + pi * RPP

            def zero(i, _):
                acc[pl.ds(i * 16, 16)] = jnp.zeros((16,), jnp.float32)
                return 0

            lax.fori_loop(0, ACC // 16, zero, 0)

            def chunk(cc32, _):
                pltpu.sync_copy(dst_hbm.at[pl.ds(cc32 * CHK, CHK)], dch)
                pltpu.sync_copy(src_hbm.at[pl.ds(cc32 * CHK, CHK)], sch)
                for k in range(K):
                    pltpu.sync_copy(
                        p_hbm.at[pl.ds(k * E_PAD + cc32 * CHK, CHK)],
                        pchs[k])

                def sel(v, cnt):
                    dv = dch[pl.ds(v * 16, 16)]
                    msk = (dv >= row_lo) & (dv < row_lo + RPP)
                    cs = plsc.cumsum(msk.astype(jnp.int32))
                    plsc.store_scatter(posb, [cnt + cs - 1],
                                       ji + v * 16, mask=msk)
                    return cnt + jnp.max(cs)

                cnt = lax.fori_loop(0, VPT, sel, 0)
                ng = (cnt + 15) // 16

                def grp(g, _):
                    pos = posb[pl.ds(g * 16, 16)]
                    valid = (g * 16 + ji) < cnt
                    pos = jnp.where(valid, pos, 0)
                    dv = plsc.load_gather(dch, [pos])
                    d_loc = jnp.where(valid, dv - row_lo, RPP)
                    dbase = d_loc * F
                    sv = plsc.load_gather(sch, [pos])
                    sv = jnp.where(valid, sv, wid)
                    cps = []
                    for j in range(16):
                        sj = jnp.sum(jnp.where(ji == j, sv, 0))
                        cps.append(pltpu.async_copy(
                            h_hbm.at[pl.ds(sj * F, F)],
                            rowb.at[pl.ds(j * F, F)], sem3))
                    alphas = [jnp.where(valid,
                                        plsc.load_gather(pchs[k], [pos]),
                                        0.0)
                              for k in range(K)]
                    for cp in cps:
                        cp.wait()
                    for j in range(16):
                        dj = jnp.max(jnp.where(ji == j, dbase, 0))
                        for k in range(K):
                            spl = jnp.sum(jnp.where(ji == j, alphas[k], 0.0))

                            def accum(cc, _, dj=dj, k=k, j=j, spl=spl):
                                off = dj + k * C + cc * 16
                                acc[pl.ds(off, 16)] = (
                                    acc[pl.ds(off, 16)]
                                    + spl * rowb[pl.ds(j * F + k * C + cc * 16,
                                                       16)])
                                return 0

                            lax.fori_loop(0, CCH, accum, 0)
                    return 0

                lax.fori_loop(0, ng, grp, 0)
                return 0

            lax.fori_loop(0, NCHK, chunk, 0)

            for k in range(K):
                pltpu.sync_copy(den_hbm.at[pl.ds(k * S + row_lo, RPP)],
                                denb.at[pl.ds(k * RPP, RPP)])

            def drain(q, _):
                for k in range(K):
                    den16 = denb[pl.ds(k * RPP + q * 16, 16)]
                    rec = 1.0 / (den16 + 1e-16)
                    for j in range(16):
                        rs = jnp.sum(jnp.where(ji == j, rec, 0.0))
                        rbase = (q * 16 + j) * F + k * C

                        def dsc(cc, _, rbase=rbase, rs=rs):
                            off = rbase + cc * 16
                            acc[pl.ds(off, 16)] = acc[pl.ds(off, 16)] * rs
                            return 0

                        lax.fori_loop(0, CCH, dsc, 0)
                return 0

            lax.fori_loop(0, RPP // 16, drain, 0)
            pltpu.sync_copy(acc.at[pl.ds(0, RPP * F)],
                            outf_hbm.at[pl.ds(row_lo * F, RPP * F)])

    return phase_e


def _node_table(cols):
    k = cols.shape[1]
    tab = jnp.zeros((k, S), jnp.float32)
    tab = tab.at[:, :N].set(cols.T)
    return tab.reshape(-1)


def _gat_edge_phase(src_p, dst_p, a_src, a_dst, h, K, F, P):
    asrc_f = _node_table(a_src)
    adst_f = _node_table(a_dst)
    m_part = _make_phase_a(K)(src_p, dst_p, asrc_f, adst_f)
    m_glob = _make_merge(K, add=False)(m_part)
    p_all, den_part = _make_phase_c(K)(src_p, dst_p, asrc_f, adst_f, m_glob)
    den_glob = _make_merge(K, add=True)(den_part)
    outf = _make_phase_e(K, F, P)(dst_p, src_p, p_all, h.reshape(-1),
                                  den_glob)
    return outf.reshape(S, F)


def kernel(x, edge_index, W1, att_src1, att_dst1, b1, W2, att_src2, att_dst2, b2, W_out, b_out):
    src = edge_index[0]
    dst = edge_index[1]
    pad = E_PAD - E
    src_p = jnp.concatenate([src, jnp.zeros((pad,), jnp.int32)])
    dst_p = jnp.concatenate([dst, jnp.full((pad,), N, jnp.int32)])

    z = jnp.zeros((HID,), jnp.float32)
    asrc1 = att_src1.reshape(HEADS, HID)
    adst1 = att_dst1.reshape(HEADS, HID)
    A1 = jnp.stack(
        [
            jnp.concatenate([asrc1[0], z]),
            jnp.concatenate([z, asrc1[1]]),
            jnp.concatenate([adst1[0], z]),
            jnp.concatenate([z, adst1[1]]),
        ],
        axis=1,
    )
    A2 = jnp.stack([att_src2.reshape(HID), att_dst2.reshape(HID)], axis=1)

    h1, att1 = _mm_att(x, W1, A1)
    out1 = _gat_edge_phase(src_p, dst_p, att1[:, :HEADS], att1[:, HEADS:],
                           h1, HEADS, HEADS * HID, 5)

    h2, att2 = _elu_mm_att(out1, b1, W2, A2)
    out2 = _gat_edge_phase(src_p, dst_p, att2[:N, :1], att2[:N, 1:],
                           h2, 1, HID, 2)

    Wb = jnp.concatenate([W_out, jnp.zeros((HID, 1), jnp.float32)], axis=1)
    y, _ = _elu_mm_att(out2, b2, Wb, jnp.zeros((2, 1), jnp.float32))
    return y[:N, :1] + b_out
